# baseline (speedup 1.0000x reference)
import functools

import jax
import jax.numpy as jnp
from jax import lax
from jax.experimental import pallas as pl
from jax.experimental.pallas import tpu as pltpu
from jax.experimental.pallas import tpu_sc as plsc

N = 10000
E = 160000
D = 256
H = 512
G = 16

NC, NS = 2, 16
NW = NC * NS
NRANGE = 64
R = 160
NPAD = NRANGE * R
R2 = 2 * R

PREP_BLK = 3200
NBLKP = E // PREP_BLK
PGRP = PREP_BLK // 16
CAPV = 8192
OVF = CAPV - PREP_BLK - 16
FL = 4096
FLF = 256
CAPR = E + FL + FLF

MC = 1024
GB = 32

_mesh = plsc.VectorSubcoreMesh(
    core_axis_name="c", subcore_axis_name="s", num_cores=NC, num_subcores=NS)
_sc_params = pltpu.CompilerParams(needs_layout_passes=False)


@functools.partial(
    pl.kernel,
    out_type=(
        jax.ShapeDtypeStruct((NPAD,), jnp.float32),
        jax.ShapeDtypeStruct((NRANGE, CAPR), jnp.int32),
        jax.ShapeDtypeStruct((NRANGE, 16), jnp.int32),
    ),
    mesh=_mesh,
    scratch_types=[
        pltpu.VMEM((PREP_BLK,), jnp.int32),
        pltpu.VMEM((CAPV,), jnp.int32),
        pltpu.VMEM((CAPV,), jnp.int32),
        pltpu.VMEM((R2,), jnp.float32),
        pltpu.VMEM((16,), jnp.int32),
    ],
    compiler_params=_sc_params,
)
def _prep_kernel(epk_hbm, deg_hbm, elist_hbm, ecnt_hbm,
                 eblk, lbufA, lbufB, degloc, cntv):
    wid = lax.axis_index("s") * NC + lax.axis_index("c")
    lo = wid * R2
    mid = lo + R
    hi = lo + R2
    rngA = 2 * wid
    rngB = rngA + 1
    ones = jnp.ones((16,), jnp.float32)
    zero = jnp.zeros((16,), jnp.float32)

    def z(i, c):
        degloc[pl.ds(i * 16, 16)] = zero
        return c
    lax.fori_loop(0, R2 // 16, z, 0)

    def overflow_flush(lbuf, rng, w, tot):
        pltpu.sync_copy(lbuf.at[pl.ds(0, FL)],
                        elist_hbm.at[rng, pl.ds(pl.multiple_of(tot, FL), FL)])
        nmv = (w - FL + 15) // 16

        def mv(i, c):
            v = lbuf[pl.ds(FL + i * 16, 16)]
            lbuf[pl.ds(i * 16, 16)] = v
            return c
        lax.fori_loop(0, nmv, mv, 0)
        return w - FL, tot + FL

    def blk(b, carry):
        wA, wB, totA, totB = carry
        pltpu.sync_copy(
            epk_hbm.at[pl.ds(pl.multiple_of(b * PREP_BLK, PREP_BLK),
                             PREP_BLK)], eblk)

        def grp(g, c):
            wA, wB = c
            e = eblk[pl.ds(g * 16, 16)]
            d = lax.shift_right_logical(e, 14)
            mA = (d >= lo) & (d < mid)
            mB = (d >= mid) & (d < hi)
            plsc.store_compressed(lbufA.at[pl.ds(wA, 16)], e, mask=mA)
            plsc.store_compressed(lbufB.at[pl.ds(wB, 16)], e, mask=mB)
            m = mA | mB
            dl = jnp.where(m, d - lo, 0)
            plsc.addupdate_scatter(degloc, [dl], ones, mask=m)
            return (wA + jnp.sum(mA.astype(jnp.int32)),
                    wB + jnp.sum(mB.astype(jnp.int32)))
        wA, wB = lax.fori_loop(0, PGRP, grp, (wA, wB))

        wA, totA = lax.cond(wA > OVF, lambda: overflow_flush(lbufA, rngA, wA, totA),
                            lambda: (wA, totA))
        wB, totB = lax.cond(wB > OVF, lambda: overflow_flush(lbufB, rngB, wB, totB),
                            lambda: (wB, totB))
        return (wA, wB, totA, totB)

    wA, wB, totA, totB = lax.fori_loop(
        0, NBLKP, blk,
        (jnp.int32(0), jnp.int32(0), jnp.int32(0), jnp.int32(0)))

    def final_flush(lbuf, rng, w, tot):
        nf = (w + FLF - 1) // FLF

        def fl(k, c):
            pltpu.sync_copy(
                lbuf.at[pl.ds(pl.multiple_of(k * FLF, FLF), FLF)],
                elist_hbm.at[rng, pl.ds(pl.multiple_of(tot + k * FLF, FLF),
                                        FLF)])
            return c
        lax.fori_loop(0, nf, fl, 0)
        cntv[...] = jnp.broadcast_to(tot + w, (16,)).astype(jnp.int32)
        pltpu.sync_copy(cntv, ecnt_hbm.at[rng])

    final_flush(lbufA, rngA, wA, totA)
    final_flush(lbufB, rngB, wB, totB)
    pltpu.sync_copy(degloc, deg_hbm.at[pl.ds(pl.multiple_of(lo, R2), R2)])


@functools.partial(
    pl.kernel,
    out_type=jax.ShapeDtypeStruct((NPAD * H,), jnp.float32),
    mesh=_mesh,
    scratch_types=[
        pltpu.VMEM((R * H,), jnp.float32),
        pltpu.VMEM((GB, H), jnp.float32),
        pltpu.VMEM((GB, H), jnp.float32),
        pltpu.VMEM((MC,), jnp.int32),
        pltpu.VMEM((MC,), jnp.int32),
        pltpu.VMEM((16,), jnp.int32),
        pltpu.SemaphoreType.DMA,
        pltpu.SemaphoreType.DMA,
    ],
    compiler_params=_sc_params,
)
def _spmm_kernel(hs_hbm, hsf_hbm, elist_hbm, ecnt_hbm, agg_hbm,
                 acc, rows0, rows1, ebuf, sidx, cntv, sem0, sem1):
    wid = lax.axis_index("s") * NC + lax.axis_index("c")
    iota = jnp.arange(16, dtype=jnp.int32)

    def do_pass(p, c0):
        rng = 2 * wid + p
        lo = pl.multiple_of(rng * R, R)
        pltpu.sync_copy(ecnt_hbm.at[rng], cntv)
        cnt = jnp.max(cntv[...])
        pltpu.sync_copy(
            hsf_hbm.at[pl.ds(pl.multiple_of(rng * (R * H), R * H), R * H)],
            acc)

        def macro(mc, c1):
            men = jnp.minimum(cnt - mc * MC, MC)
            pltpu.sync_copy(
                elist_hbm.at[rng, pl.ds(pl.multiple_of(mc * MC, MC), MC)],
                ebuf)

            def unpack(g, c2):
                e = ebuf[pl.ds(g * 16, 16)]
                valid = (g * 16 + iota) < men
                s = jnp.where(valid, e & 16383, 0)
                sidx[pl.ds(g * 16, 16)] = s
                return c2
            lax.fori_loop(0, MC // 16, unpack, 0, unroll=4)

            nb = (men + GB - 1) // GB

            def fire(k, rows, sem):
                pltpu.async_copy(
                    hs_hbm.at[sidx.at[pl.ds(pl.multiple_of(k * GB, GB), GB)]],
                    rows, sem)

            def wait(rows, sem):
                pltpu.make_async_copy(
                    hs_hbm.at[pl.ds(0, GB)], rows, sem).wait()

            def accum(k, rows):
                base = k * GB

                def half(boff):
                    e = ebuf[pl.ds(base + boff, 16)]
                    abv = (lax.shift_right_logical(e, 14) - lo) * H

                    def edge(j, c):
                        jj = base + boff + j
                        ab = pl.multiple_of(
                            jnp.max(jnp.where(iota == j, abv, 0)), H)
                        jv = jnp.broadcast_to(boff + j, (16,)).astype(
                            jnp.int32)

                        @pl.when(jj < men)
                        def _():
                            for cb in range(H // 16):
                                v = plsc.load_gather(
                                    rows, [jv, cb * 16 + iota])
                                asl = pl.ds(ab + cb * 16, 16)
                                acc[asl] = acc[asl] + v
                        return c
                    lax.fori_loop(0, 16, edge, 0)
                half(0)
                half(16)

            @pl.when(nb > 0)
            def _():
                fire(0, rows0, sem0)

                def pair(q, c3):
                    k0 = 2 * q
                    k1 = k0 + 1

                    @pl.when(k1 < nb)
                    def _():
                        fire(k1, rows1, sem1)
                    wait(rows0, sem0)
                    accum(k0, rows0)

                    @pl.when(k1 < nb)
                    def _():
                        @pl.when(k1 + 1 < nb)
                        def _():
                            fire(k1 + 1, rows0, sem0)
                        wait(rows1, sem1)
                        accum(k1, rows1)
                    return c3
                lax.fori_loop(0, (nb + 1) // 2, pair, 0)
            return c1
        lax.fori_loop(0, (cnt + MC - 1) // MC, macro, 0)

        pltpu.sync_copy(
            acc,
            agg_hbm.at[pl.ds(pl.multiple_of(rng * (R * H), R * H), R * H)])
        return c0
    lax.fori_loop(0, 2, do_pass, 0)


BM = 256
GRID = NPAD // BM


def _tc1_body(x_ref, w_ref, deg_ref, hs_ref):
    dis = lax.rsqrt(deg_ref[...] + 1.0)
    hs_ref[...] = jnp.dot(x_ref[...], w_ref[...],
                          preferred_element_type=jnp.float32) * dis


_tc1 = pl.pallas_call(
    _tc1_body,
    grid=(GRID,),
    in_specs=[
        pl.BlockSpec((BM, D), lambda i: (i, 0)),
        pl.BlockSpec((D, H), lambda i: (0, 0)),
        pl.BlockSpec((BM, 1), lambda i: (i, 0)),
    ],
    out_specs=pl.BlockSpec((BM, H), lambda i: (i, 0)),
    out_shape=jax.ShapeDtypeStruct((NPAD, H), jnp.float32),
)


def _tcmid_body(agg_ref, deg_ref, b_ref, w_ref, hs_ref):
    dis = lax.rsqrt(deg_ref[...] + 1.0)
    o = jnp.maximum(agg_ref[...] * dis + b_ref[...], 0.0)
    hs_ref[...] = jnp.dot(o, w_ref[...],
                          preferred_element_type=jnp.float32) * dis


_tcmid = pl.pallas_call(
    _tcmid_body,
    grid=(GRID,),
    in_specs=[
        pl.BlockSpec((BM, H), lambda i: (i, 0)),
        pl.BlockSpec((BM, 1), lambda i: (i, 0)),
        pl.BlockSpec((1, H), lambda i: (0, 0)),
        pl.BlockSpec((H, H), lambda i: (0, 0)),
    ],
    out_specs=pl.BlockSpec((BM, H), lambda i: (i, 0)),
    out_shape=jax.ShapeDtypeStruct((NPAD, H), jnp.float32),
)


def _tc4_body(agg_ref, deg_ref, b_ref, batch_ref, sums_ref, cnt_ref):
    i = pl.program_id(0)
    dis = lax.rsqrt(deg_ref[...] + 1.0)
    o = agg_ref[...] * dis + b_ref[...]
    oh = (batch_ref[...] == lax.broadcasted_iota(jnp.int32, (1, G), 1))
    oh = oh.astype(jnp.float32)
    ps = jnp.dot(oh.T, o, preferred_element_type=jnp.float32)
    pc = jnp.sum(oh, axis=0)[:, None]

    @pl.when(i == 0)
    def _():
        sums_ref[...] = jnp.zeros_like(sums_ref)
        cnt_ref[...] = jnp.zeros_like(cnt_ref)

    sums_ref[...] += ps
    cnt_ref[...] += jnp.broadcast_to(pc, (G, 128))


_tc4 = pl.pallas_call(
    _tc4_body,
    grid=(GRID,),
    in_specs=[
        pl.BlockSpec((BM, H), lambda i: (i, 0)),
        pl.BlockSpec((BM, 1), lambda i: (i, 0)),
        pl.BlockSpec((1, H), lambda i: (0, 0)),
        pl.BlockSpec((BM, 1), lambda i: (i, 0)),
    ],
    out_specs=(
        pl.BlockSpec((G, H), lambda i: (0, 0)),
        pl.BlockSpec((G, 128), lambda i: (0, 0)),
    ),
    out_shape=(
        jax.ShapeDtypeStruct((G, H), jnp.float32),
        jax.ShapeDtypeStruct((G, 128), jnp.float32),
    ),
)


def _tc5_body(sums_ref, cnt_ref, wl_ref, bl_ref, logits_ref, probs_ref):
    cnt = jnp.maximum(cnt_ref[...][:, 0:1], 1.0)
    pooled = sums_ref[...] / cnt
    logits = jnp.dot(pooled, wl_ref[...],
                     preferred_element_type=jnp.float32) + bl_ref[...]
    logits_ref[...] = logits
    mx = jnp.max(logits, axis=-1, keepdims=True)
    e = jnp.exp(logits - mx)
    probs_ref[...] = e / jnp.sum(e, axis=-1, keepdims=True)


def _tc5(sums, cnt, Wl, bl):
    C = Wl.shape[1]
    return pl.pallas_call(
        _tc5_body,
        out_shape=(
            jax.ShapeDtypeStruct((G, C), jnp.float32),
            jax.ShapeDtypeStruct((G, C), jnp.float32),
        ),
    )(sums, cnt, Wl, bl)


def kernel(x, edge_index, batch, W1, b1, W2, b2, W3, b3, Wl, bl):
    src = edge_index[0]
    dst = edge_index[1]
    epk = jnp.bitwise_or(src, jnp.left_shift(dst, 14))
    xp = jnp.pad(x, ((0, NPAD - N), (0, 0)))
    batchp = jnp.pad(batch, (0, NPAD - N), constant_values=G).reshape(NPAD, 1)
    deg, elist, ecnt = _prep_kernel(epk)
    deg = deg.reshape(NPAD, 1)

    def spmm(hs):
        return _spmm_kernel(hs, hs.reshape(-1), elist, ecnt).reshape(NPAD, H)

    hs1 = _tc1(xp, W1, deg)
    agg1 = spmm(hs1)
    hs2 = _tcmid(agg1, deg, b1.reshape(1, H), W2)
    agg2 = spmm(hs2)
    hs3 = _tcmid(agg2, deg, b2.reshape(1, H), W3)
    agg3 = spmm(hs3)
    sums, cnt = _tc4(agg3, deg, b3.reshape(1, H), batchp)
    logits, probs = _tc5(sums, cnt, Wl, bl.reshape(1, -1))
    return (logits, probs)


# speedup vs baseline: 1.5123x; 1.5123x over previous
import functools

import jax
import jax.numpy as jnp
from jax import lax
from jax.experimental import pallas as pl
from jax.experimental.pallas import tpu as pltpu
from jax.experimental.pallas import tpu_sc as plsc

N = 10000
E = 160000
D = 256
H = 512
G = 16

NC, NS = 2, 16
NW = NC * NS
NRANGE = 64
R = 160
NPAD = NRANGE * R
R2 = 2 * R

PREP_BLK = 3200
NBLKP = E // PREP_BLK
PGRP = PREP_BLK // 16
CAPV = 8192
OVF = CAPV - PREP_BLK - 16
FL = 4096
FLF = 256
CAPR = E + FL + FLF

MC = 1024
GB = 32

_mesh = plsc.VectorSubcoreMesh(
    core_axis_name="c", subcore_axis_name="s", num_cores=NC, num_subcores=NS)
_sc_params = pltpu.CompilerParams(needs_layout_passes=False)


@functools.partial(
    pl.kernel,
    out_type=(
        jax.ShapeDtypeStruct((NPAD,), jnp.float32),
        jax.ShapeDtypeStruct((NRANGE, CAPR), jnp.int32),
        jax.ShapeDtypeStruct((NRANGE, 16), jnp.int32),
    ),
    mesh=_mesh,
    scratch_types=[
        pltpu.VMEM((PREP_BLK,), jnp.int32),
        pltpu.VMEM((CAPV,), jnp.int32),
        pltpu.VMEM((CAPV,), jnp.int32),
        pltpu.VMEM((R2,), jnp.float32),
        pltpu.VMEM((16,), jnp.int32),
    ],
    compiler_params=_sc_params,
)
def _prep_kernel(epk_hbm, deg_hbm, elist_hbm, ecnt_hbm,
                 eblk, lbufA, lbufB, degloc, cntv):
    wid = lax.axis_index("s") * NC + lax.axis_index("c")
    lo = wid * R2
    mid = lo + R
    hi = lo + R2
    rngA = 2 * wid
    rngB = rngA + 1
    ones = jnp.ones((16,), jnp.float32)
    zero = jnp.zeros((16,), jnp.float32)

    def z(i, c):
        degloc[pl.ds(i * 16, 16)] = zero
        return c
    lax.fori_loop(0, R2 // 16, z, 0)

    def overflow_flush(lbuf, rng, w, tot):
        pltpu.sync_copy(lbuf.at[pl.ds(0, FL)],
                        elist_hbm.at[rng, pl.ds(pl.multiple_of(tot, FL), FL)])
        nmv = (w - FL + 15) // 16

        def mv(i, c):
            v = lbuf[pl.ds(FL + i * 16, 16)]
            lbuf[pl.ds(i * 16, 16)] = v
            return c
        lax.fori_loop(0, nmv, mv, 0)
        return w - FL, tot + FL

    def blk(b, carry):
        wA, wB, totA, totB = carry
        pltpu.sync_copy(
            epk_hbm.at[pl.ds(pl.multiple_of(b * PREP_BLK, PREP_BLK),
                             PREP_BLK)], eblk)

        def grp(g, c):
            wA, wB = c
            e = eblk[pl.ds(g * 16, 16)]
            d = lax.shift_right_logical(e, 14)
            mA = (d >= lo) & (d < mid)
            mB = (d >= mid) & (d < hi)
            plsc.store_compressed(lbufA.at[pl.ds(wA, 16)], e, mask=mA)
            plsc.store_compressed(lbufB.at[pl.ds(wB, 16)], e, mask=mB)
            m = mA | mB
            dl = jnp.where(m, d - lo, 0)
            plsc.addupdate_scatter(degloc, [dl], ones, mask=m)
            return (wA + jnp.sum(mA.astype(jnp.int32)),
                    wB + jnp.sum(mB.astype(jnp.int32)))
        wA, wB = lax.fori_loop(0, PGRP, grp, (wA, wB))

        wA, totA = lax.cond(wA > OVF, lambda: overflow_flush(lbufA, rngA, wA, totA),
                            lambda: (wA, totA))
        wB, totB = lax.cond(wB > OVF, lambda: overflow_flush(lbufB, rngB, wB, totB),
                            lambda: (wB, totB))
        return (wA, wB, totA, totB)

    wA, wB, totA, totB = lax.fori_loop(
        0, NBLKP, blk,
        (jnp.int32(0), jnp.int32(0), jnp.int32(0), jnp.int32(0)))

    def final_flush(lbuf, rng, w, tot):
        nf = (w + FLF - 1) // FLF

        def fl(k, c):
            pltpu.sync_copy(
                lbuf.at[pl.ds(pl.multiple_of(k * FLF, FLF), FLF)],
                elist_hbm.at[rng, pl.ds(pl.multiple_of(tot + k * FLF, FLF),
                                        FLF)])
            return c
        lax.fori_loop(0, nf, fl, 0)
        cntv[...] = jnp.broadcast_to(tot + w, (16,)).astype(jnp.int32)
        pltpu.sync_copy(cntv, ecnt_hbm.at[rng])

    final_flush(lbufA, rngA, wA, totA)
    final_flush(lbufB, rngB, wB, totB)
    pltpu.sync_copy(degloc, deg_hbm.at[pl.ds(pl.multiple_of(lo, R2), R2)])


@functools.partial(
    pl.kernel,
    out_type=jax.ShapeDtypeStruct((NPAD * H,), jnp.float32),
    mesh=_mesh,
    scratch_types=[
        pltpu.VMEM((R * H,), jnp.float32),
        pltpu.VMEM((GB, H), jnp.float32),
        pltpu.VMEM((GB, H), jnp.float32),
        pltpu.VMEM((MC,), jnp.int32),
        pltpu.VMEM((MC,), jnp.int32),
        pltpu.VMEM((16,), jnp.int32),
        pltpu.SemaphoreType.DMA,
        pltpu.SemaphoreType.DMA,
    ],
    compiler_params=_sc_params,
)
def _spmm_kernel(hs_hbm, hsf_hbm, elist_hbm, ecnt_hbm, agg_hbm,
                 acc, rows0, rows1, ebuf, sidx, cntv, sem0, sem1):
    wid = lax.axis_index("s") * NC + lax.axis_index("c")
    iota = jnp.arange(16, dtype=jnp.int32)

    def do_pass(p, c0):
        rng = 2 * wid + p
        lo = pl.multiple_of(rng * R, R)
        pltpu.sync_copy(ecnt_hbm.at[rng], cntv)
        cnt = jnp.max(cntv[...])
        pltpu.sync_copy(
            hsf_hbm.at[pl.ds(pl.multiple_of(rng * (R * H), R * H), R * H)],
            acc)

        def macro(mc, c1):
            men = jnp.minimum(cnt - mc * MC, MC)
            pltpu.sync_copy(
                elist_hbm.at[rng, pl.ds(pl.multiple_of(mc * MC, MC), MC)],
                ebuf)

            def unpack(g, c2):
                e = ebuf[pl.ds(g * 16, 16)]
                valid = (g * 16 + iota) < men
                s = jnp.where(valid, e & 16383, 0)
                sidx[pl.ds(g * 16, 16)] = s
                return c2
            lax.fori_loop(0, MC // 16, unpack, 0, unroll=4)

            nb = (men + GB - 1) // GB

            def fire(k, rows, sem):
                pltpu.async_copy(
                    hs_hbm.at[sidx.at[pl.ds(pl.multiple_of(k * GB, GB), GB)]],
                    rows, sem)

            def wait(rows, sem):
                pltpu.make_async_copy(
                    hs_hbm.at[pl.ds(0, GB)], rows, sem).wait()

            def accum(k, rows):
                base = k * GB

                def half(boff):
                    e = ebuf[pl.ds(base + boff, 16)]
                    abv = (lax.shift_right_logical(e, 14) - lo) * H

                    def edge(j, c):
                        jj = base + boff + j
                        ab = pl.multiple_of(
                            jnp.max(jnp.where(iota == j, abv, 0)), H)
                        jv = jnp.broadcast_to(boff + j, (16,)).astype(
                            jnp.int32)

                        @pl.when(jj < men)
                        def _():
                            nch = H // 16
                            vs = [plsc.load_gather(rows, [jv, cb * 16 + iota])
                                  for cb in range(nch)]
                            avs = [acc[pl.ds(ab + cb * 16, 16)]
                                   for cb in range(nch)]
                            for cb in range(nch):
                                acc[pl.ds(ab + cb * 16, 16)] = avs[cb] + vs[cb]
                        return c
                    lax.fori_loop(0, 16, edge, 0)
                half(0)
                half(16)

            @pl.when(nb > 0)
            def _():
                fire(0, rows0, sem0)

                def pair(q, c3):
                    k0 = 2 * q
                    k1 = k0 + 1

                    @pl.when(k1 < nb)
                    def _():
                        fire(k1, rows1, sem1)
                    wait(rows0, sem0)
                    accum(k0, rows0)

                    @pl.when(k1 < nb)
                    def _():
                        @pl.when(k1 + 1 < nb)
                        def _():
                            fire(k1 + 1, rows0, sem0)
                        wait(rows1, sem1)
                        accum(k1, rows1)
                    return c3
                lax.fori_loop(0, (nb + 1) // 2, pair, 0)
            return c1
        lax.fori_loop(0, (cnt + MC - 1) // MC, macro, 0)

        pltpu.sync_copy(
            acc,
            agg_hbm.at[pl.ds(pl.multiple_of(rng * (R * H), R * H), R * H)])
        return c0
    lax.fori_loop(0, 2, do_pass, 0)


BM = 256
GRID = NPAD // BM


def _tc1_body(x_ref, w_ref, deg_ref, hs_ref):
    dis = lax.rsqrt(deg_ref[...] + 1.0)
    hs_ref[...] = jnp.dot(x_ref[...], w_ref[...],
                          preferred_element_type=jnp.float32) * dis


_tc1 = pl.pallas_call(
    _tc1_body,
    grid=(GRID,),
    in_specs=[
        pl.BlockSpec((BM, D), lambda i: (i, 0)),
        pl.BlockSpec((D, H), lambda i: (0, 0)),
        pl.BlockSpec((BM, 1), lambda i: (i, 0)),
    ],
    out_specs=pl.BlockSpec((BM, H), lambda i: (i, 0)),
    out_shape=jax.ShapeDtypeStruct((NPAD, H), jnp.float32),
)


def _tcmid_body(agg_ref, deg_ref, b_ref, w_ref, hs_ref):
    dis = lax.rsqrt(deg_ref[...] + 1.0)
    o = jnp.maximum(agg_ref[...] * dis + b_ref[...], 0.0)
    hs_ref[...] = jnp.dot(o, w_ref[...],
                          preferred_element_type=jnp.float32) * dis


_tcmid = pl.pallas_call(
    _tcmid_body,
    grid=(GRID,),
    in_specs=[
        pl.BlockSpec((BM, H), lambda i: (i, 0)),
        pl.BlockSpec((BM, 1), lambda i: (i, 0)),
        pl.BlockSpec((1, H), lambda i: (0, 0)),
        pl.BlockSpec((H, H), lambda i: (0, 0)),
    ],
    out_specs=pl.BlockSpec((BM, H), lambda i: (i, 0)),
    out_shape=jax.ShapeDtypeStruct((NPAD, H), jnp.float32),
)


def _tc4_body(agg_ref, deg_ref, b_ref, batch_ref, sums_ref, cnt_ref):
    i = pl.program_id(0)
    dis = lax.rsqrt(deg_ref[...] + 1.0)
    o = agg_ref[...] * dis + b_ref[...]
    oh = (batch_ref[...] == lax.broadcasted_iota(jnp.int32, (1, G), 1))
    oh = oh.astype(jnp.float32)
    ps = jnp.dot(oh.T, o, preferred_element_type=jnp.float32)
    pc = jnp.sum(oh, axis=0)[:, None]

    @pl.when(i == 0)
    def _():
        sums_ref[...] = jnp.zeros_like(sums_ref)
        cnt_ref[...] = jnp.zeros_like(cnt_ref)

    sums_ref[...] += ps
    cnt_ref[...] += jnp.broadcast_to(pc, (G, 128))


_tc4 = pl.pallas_call(
    _tc4_body,
    grid=(GRID,),
    in_specs=[
        pl.BlockSpec((BM, H), lambda i: (i, 0)),
        pl.BlockSpec((BM, 1), lambda i: (i, 0)),
        pl.BlockSpec((1, H), lambda i: (0, 0)),
        pl.BlockSpec((BM, 1), lambda i: (i, 0)),
    ],
    out_specs=(
        pl.BlockSpec((G, H), lambda i: (0, 0)),
        pl.BlockSpec((G, 128), lambda i: (0, 0)),
    ),
    out_shape=(
        jax.ShapeDtypeStruct((G, H), jnp.float32),
        jax.ShapeDtypeStruct((G, 128), jnp.float32),
    ),
)


def _tc5_body(sums_ref, cnt_ref, wl_ref, bl_ref, logits_ref, probs_ref):
    cnt = jnp.maximum(cnt_ref[...][:, 0:1], 1.0)
    pooled = sums_ref[...] / cnt
    logits = jnp.dot(pooled, wl_ref[...],
                     preferred_element_type=jnp.float32) + bl_ref[...]
    logits_ref[...] = logits
    mx = jnp.max(logits, axis=-1, keepdims=True)
    e = jnp.exp(logits - mx)
    probs_ref[...] = e / jnp.sum(e, axis=-1, keepdims=True)


def _tc5(sums, cnt, Wl, bl):
    C = Wl.shape[1]
    return pl.pallas_call(
        _tc5_body,
        out_shape=(
            jax.ShapeDtypeStruct((G, C), jnp.float32),
            jax.ShapeDtypeStruct((G, C), jnp.float32),
        ),
    )(sums, cnt, Wl, bl)


def kernel(x, edge_index, batch, W1, b1, W2, b2, W3, b3, Wl, bl):
    src = edge_index[0]
    dst = edge_index[1]
    epk = jnp.bitwise_or(src, jnp.left_shift(dst, 14))
    xp = jnp.pad(x, ((0, NPAD - N), (0, 0)))
    batchp = jnp.pad(batch, (0, NPAD - N), constant_values=G).reshape(NPAD, 1)
    deg, elist, ecnt = _prep_kernel(epk)
    deg = deg.reshape(NPAD, 1)

    def spmm(hs):
        return _spmm_kernel(hs, hs.reshape(-1), elist, ecnt).reshape(NPAD, H)

    hs1 = _tc1(xp, W1, deg)
    agg1 = spmm(hs1)
    hs2 = _tcmid(agg1, deg, b1.reshape(1, H), W2)
    agg2 = spmm(hs2)
    hs3 = _tcmid(agg2, deg, b2.reshape(1, H), W3)
    agg3 = spmm(hs3)
    sums, cnt = _tc4(agg3, deg, b3.reshape(1, H), batchp)
    logits, probs = _tc5(sums, cnt, Wl, bl.reshape(1, -1))
    return (logits, probs)


# speedup vs baseline: 1.5629x; 1.0335x over previous
import functools

import jax
import jax.numpy as jnp
from jax import lax
from jax.experimental import pallas as pl
from jax.experimental.pallas import tpu as pltpu
from jax.experimental.pallas import tpu_sc as plsc

N = 10000
E = 160000
D = 256
H = 512
G = 16

NC, NS = 2, 16
NW = NC * NS
NRANGE = 64
R = 160
NPAD = NRANGE * R
R2 = 2 * R

PREP_BLK = 3200
NBLKP = E // PREP_BLK
PGRP = PREP_BLK // 16
CAPV = 8192
OVF = CAPV - PREP_BLK - 16
FL = 4096
FLF = 256
CAPR = E + FL + FLF

MC = 1024
GB = 32

_mesh = plsc.VectorSubcoreMesh(
    core_axis_name="c", subcore_axis_name="s", num_cores=NC, num_subcores=NS)
_sc_params = pltpu.CompilerParams(needs_layout_passes=False)


@functools.partial(
    pl.kernel,
    out_type=(
        jax.ShapeDtypeStruct((NPAD,), jnp.float32),
        jax.ShapeDtypeStruct((NRANGE, CAPR), jnp.int32),
        jax.ShapeDtypeStruct((NRANGE, 16), jnp.int32),
    ),
    mesh=_mesh,
    scratch_types=[
        pltpu.VMEM((PREP_BLK,), jnp.int32),
        pltpu.VMEM((CAPV,), jnp.int32),
        pltpu.VMEM((CAPV,), jnp.int32),
        pltpu.VMEM((R2,), jnp.float32),
        pltpu.VMEM((16,), jnp.int32),
    ],
    compiler_params=_sc_params,
)
def _prep_kernel(epk_hbm, deg_hbm, elist_hbm, ecnt_hbm,
                 eblk, lbufA, lbufB, degloc, cntv):
    wid = lax.axis_index("s") * NC + lax.axis_index("c")
    lo = wid * R2
    mid = lo + R
    hi = lo + R2
    rngA = 2 * wid
    rngB = rngA + 1
    ones = jnp.ones((16,), jnp.float32)
    zero = jnp.zeros((16,), jnp.float32)

    def z(i, c):
        degloc[pl.ds(i * 16, 16)] = zero
        return c
    lax.fori_loop(0, R2 // 16, z, 0)

    def overflow_flush(lbuf, rng, w, tot):
        pltpu.sync_copy(lbuf.at[pl.ds(0, FL)],
                        elist_hbm.at[rng, pl.ds(pl.multiple_of(tot, FL), FL)])
        nmv = (w - FL + 15) // 16

        def mv(i, c):
            v = lbuf[pl.ds(FL + i * 16, 16)]
            lbuf[pl.ds(i * 16, 16)] = v
            return c
        lax.fori_loop(0, nmv, mv, 0)
        return w - FL, tot + FL

    def blk(b, carry):
        wA, wB, totA, totB = carry
        pltpu.sync_copy(
            epk_hbm.at[pl.ds(pl.multiple_of(b * PREP_BLK, PREP_BLK),
                             PREP_BLK)], eblk)

        def grp(g, c):
            wA, wB = c
            e = eblk[pl.ds(g * 16, 16)]
            d = lax.shift_right_logical(e, 14)
            mA = (d >= lo) & (d < mid)
            mB = (d >= mid) & (d < hi)
            plsc.store_compressed(lbufA.at[pl.ds(wA, 16)], e, mask=mA)
            plsc.store_compressed(lbufB.at[pl.ds(wB, 16)], e, mask=mB)
            m = mA | mB
            dl = jnp.where(m, d - lo, 0)
            plsc.addupdate_scatter(degloc, [dl], ones, mask=m)
            return (wA + jnp.sum(mA.astype(jnp.int32)),
                    wB + jnp.sum(mB.astype(jnp.int32)))
        wA, wB = lax.fori_loop(0, PGRP, grp, (wA, wB))

        wA, totA = lax.cond(wA > OVF, lambda: overflow_flush(lbufA, rngA, wA, totA),
                            lambda: (wA, totA))
        wB, totB = lax.cond(wB > OVF, lambda: overflow_flush(lbufB, rngB, wB, totB),
                            lambda: (wB, totB))
        return (wA, wB, totA, totB)

    wA, wB, totA, totB = lax.fori_loop(
        0, NBLKP, blk,
        (jnp.int32(0), jnp.int32(0), jnp.int32(0), jnp.int32(0)))

    def final_flush(lbuf, rng, w, tot):
        nf = (w + FLF - 1) // FLF

        def fl(k, c):
            pltpu.sync_copy(
                lbuf.at[pl.ds(pl.multiple_of(k * FLF, FLF), FLF)],
                elist_hbm.at[rng, pl.ds(pl.multiple_of(tot + k * FLF, FLF),
                                        FLF)])
            return c
        lax.fori_loop(0, nf, fl, 0)
        cntv[...] = jnp.broadcast_to(tot + w, (16,)).astype(jnp.int32)
        pltpu.sync_copy(cntv, ecnt_hbm.at[rng])

    final_flush(lbufA, rngA, wA, totA)
    final_flush(lbufB, rngB, wB, totB)
    pltpu.sync_copy(degloc, deg_hbm.at[pl.ds(pl.multiple_of(lo, R2), R2)])


@functools.partial(
    pl.kernel,
    out_type=jax.ShapeDtypeStruct((NPAD * H,), jnp.float32),
    mesh=_mesh,
    scratch_types=[
        pltpu.VMEM((R * H,), jnp.float32),
        pltpu.VMEM((GB, H), jnp.float32),
        pltpu.VMEM((GB, H), jnp.float32),
        pltpu.VMEM((MC,), jnp.int32),
        pltpu.VMEM((MC,), jnp.int32),
        pltpu.VMEM((16,), jnp.int32),
        pltpu.SemaphoreType.DMA,
        pltpu.SemaphoreType.DMA,
    ],
    compiler_params=_sc_params,
)
def _spmm_kernel(hs_hbm, hsf_hbm, elist_hbm, ecnt_hbm, agg_hbm,
                 acc, rows0, rows1, ebuf, sidx, cntv, sem0, sem1):
    wid = lax.axis_index("s") * NC + lax.axis_index("c")
    iota = jnp.arange(16, dtype=jnp.int32)

    def do_pass(p, c0):
        rng = 2 * wid + p
        lo = pl.multiple_of(rng * R, R)
        pltpu.sync_copy(ecnt_hbm.at[rng], cntv)
        cnt = jnp.max(cntv[...])
        pltpu.sync_copy(
            hsf_hbm.at[pl.ds(pl.multiple_of(rng * (R * H), R * H), R * H)],
            acc)

        def macro(mc, c1):
            men = jnp.minimum(cnt - mc * MC, MC)
            pltpu.sync_copy(
                elist_hbm.at[rng, pl.ds(pl.multiple_of(mc * MC, MC), MC)],
                ebuf)

            def unpack(g, c2):
                e = ebuf[pl.ds(g * 16, 16)]
                valid = (g * 16 + iota) < men
                s = jnp.where(valid, e & 16383, 0)
                sidx[pl.ds(g * 16, 16)] = s
                return c2
            lax.fori_loop(0, MC // 16, unpack, 0, unroll=4)

            nb = (men + GB - 1) // GB

            def fire(k, rows, sem):
                pltpu.async_copy(
                    hs_hbm.at[sidx.at[pl.ds(pl.multiple_of(k * GB, GB), GB)]],
                    rows, sem)

            def wait(rows, sem):
                pltpu.make_async_copy(
                    hs_hbm.at[pl.ds(0, GB)], rows, sem).wait()

            def accum(k, rows):
                base = k * GB

                def half(boff):
                    e = ebuf[pl.ds(base + boff, 16)]
                    abv = (lax.shift_right_logical(e, 14) - lo) * H

                    def edge(j, c):
                        jj = base + boff + j
                        ab = pl.multiple_of(
                            jnp.max(jnp.where(iota == j, abv, 0)), H)
                        jv = jnp.broadcast_to(boff + j, (16,)).astype(
                            jnp.int32)

                        @pl.when(jj < men)
                        def _():
                            nch = H // 16
                            vs = [plsc.load_gather(rows, [jv, cb * 16 + iota])
                                  for cb in range(nch)]
                            avs = [acc[pl.ds(ab + cb * 16, 16)]
                                   for cb in range(nch)]
                            for cb in range(nch):
                                acc[pl.ds(ab + cb * 16, 16)] = avs[cb] + vs[cb]
                        return c
                    lax.fori_loop(0, 16, edge, 0, unroll=2)
                half(0)
                half(16)

            @pl.when(nb > 0)
            def _():
                fire(0, rows0, sem0)

                def pair(q, c3):
                    k0 = 2 * q
                    k1 = k0 + 1

                    @pl.when(k1 < nb)
                    def _():
                        fire(k1, rows1, sem1)
                    wait(rows0, sem0)
                    accum(k0, rows0)

                    @pl.when(k1 < nb)
                    def _():
                        @pl.when(k1 + 1 < nb)
                        def _():
                            fire(k1 + 1, rows0, sem0)
                        wait(rows1, sem1)
                        accum(k1, rows1)
                    return c3
                lax.fori_loop(0, (nb + 1) // 2, pair, 0)
            return c1
        lax.fori_loop(0, (cnt + MC - 1) // MC, macro, 0)

        pltpu.sync_copy(
            acc,
            agg_hbm.at[pl.ds(pl.multiple_of(rng * (R * H), R * H), R * H)])
        return c0
    lax.fori_loop(0, 2, do_pass, 0)


BM = 256
GRID = NPAD // BM


def _tc1_body(x_ref, w_ref, deg_ref, hs_ref):
    dis = lax.rsqrt(deg_ref[...] + 1.0)
    hs_ref[...] = jnp.dot(x_ref[...], w_ref[...],
                          preferred_element_type=jnp.float32) * dis


_tc1 = pl.pallas_call(
    _tc1_body,
    grid=(GRID,),
    in_specs=[
        pl.BlockSpec((BM, D), lambda i: (i, 0)),
        pl.BlockSpec((D, H), lambda i: (0, 0)),
        pl.BlockSpec((BM, 1), lambda i: (i, 0)),
    ],
    out_specs=pl.BlockSpec((BM, H), lambda i: (i, 0)),
    out_shape=jax.ShapeDtypeStruct((NPAD, H), jnp.float32),
)


def _tcmid_body(agg_ref, deg_ref, b_ref, w_ref, hs_ref):
    dis = lax.rsqrt(deg_ref[...] + 1.0)
    o = jnp.maximum(agg_ref[...] * dis + b_ref[...], 0.0)
    hs_ref[...] = jnp.dot(o, w_ref[...],
                          preferred_element_type=jnp.float32) * dis


_tcmid = pl.pallas_call(
    _tcmid_body,
    grid=(GRID,),
    in_specs=[
        pl.BlockSpec((BM, H), lambda i: (i, 0)),
        pl.BlockSpec((BM, 1), lambda i: (i, 0)),
        pl.BlockSpec((1, H), lambda i: (0, 0)),
        pl.BlockSpec((H, H), lambda i: (0, 0)),
    ],
    out_specs=pl.BlockSpec((BM, H), lambda i: (i, 0)),
    out_shape=jax.ShapeDtypeStruct((NPAD, H), jnp.float32),
)


def _tc4_body(agg_ref, deg_ref, b_ref, batch_ref, sums_ref, cnt_ref):
    i = pl.program_id(0)
    dis = lax.rsqrt(deg_ref[...] + 1.0)
    o = agg_ref[...] * dis + b_ref[...]
    oh = (batch_ref[...] == lax.broadcasted_iota(jnp.int32, (1, G), 1))
    oh = oh.astype(jnp.float32)
    ps = jnp.dot(oh.T, o, preferred_element_type=jnp.float32)
    pc = jnp.sum(oh, axis=0)[:, None]

    @pl.when(i == 0)
    def _():
        sums_ref[...] = jnp.zeros_like(sums_ref)
        cnt_ref[...] = jnp.zeros_like(cnt_ref)

    sums_ref[...] += ps
    cnt_ref[...] += jnp.broadcast_to(pc, (G, 128))


_tc4 = pl.pallas_call(
    _tc4_body,
    grid=(GRID,),
    in_specs=[
        pl.BlockSpec((BM, H), lambda i: (i, 0)),
        pl.BlockSpec((BM, 1), lambda i: (i, 0)),
        pl.BlockSpec((1, H), lambda i: (0, 0)),
        pl.BlockSpec((BM, 1), lambda i: (i, 0)),
    ],
    out_specs=(
        pl.BlockSpec((G, H), lambda i: (0, 0)),
        pl.BlockSpec((G, 128), lambda i: (0, 0)),
    ),
    out_shape=(
        jax.ShapeDtypeStruct((G, H), jnp.float32),
        jax.ShapeDtypeStruct((G, 128), jnp.float32),
    ),
)


def _tc5_body(sums_ref, cnt_ref, wl_ref, bl_ref, logits_ref, probs_ref):
    cnt = jnp.maximum(cnt_ref[...][:, 0:1], 1.0)
    pooled = sums_ref[...] / cnt
    logits = jnp.dot(pooled, wl_ref[...],
                     preferred_element_type=jnp.float32) + bl_ref[...]
    logits_ref[...] = logits
    mx = jnp.max(logits, axis=-1, keepdims=True)
    e = jnp.exp(logits - mx)
    probs_ref[...] = e / jnp.sum(e, axis=-1, keepdims=True)


def _tc5(sums, cnt, Wl, bl):
    C = Wl.shape[1]
    return pl.pallas_call(
        _tc5_body,
        out_shape=(
            jax.ShapeDtypeStruct((G, C), jnp.float32),
            jax.ShapeDtypeStruct((G, C), jnp.float32),
        ),
    )(sums, cnt, Wl, bl)


def kernel(x, edge_index, batch, W1, b1, W2, b2, W3, b3, Wl, bl):
    src = edge_index[0]
    dst = edge_index[1]
    epk = jnp.bitwise_or(src, jnp.left_shift(dst, 14))
    xp = jnp.pad(x, ((0, NPAD - N), (0, 0)))
    batchp = jnp.pad(batch, (0, NPAD - N), constant_values=G).reshape(NPAD, 1)
    deg, elist, ecnt = _prep_kernel(epk)
    deg = deg.reshape(NPAD, 1)

    def spmm(hs):
        return _spmm_kernel(hs, hs.reshape(-1), elist, ecnt).reshape(NPAD, H)

    hs1 = _tc1(xp, W1, deg)
    agg1 = spmm(hs1)
    hs2 = _tcmid(agg1, deg, b1.reshape(1, H), W2)
    agg2 = spmm(hs2)
    hs3 = _tcmid(agg2, deg, b2.reshape(1, H), W3)
    agg3 = spmm(hs3)
    sums, cnt = _tc4(agg3, deg, b3.reshape(1, H), batchp)
    logits, probs = _tc5(sums, cnt, Wl, bl.reshape(1, -1))
    return (logits, probs)
